# baseline (speedup 1.0000x reference)
import jax
import jax.numpy as jnp
from jax.experimental import pallas as pl
from jax.experimental.pallas import tpu as pltpu

_TAU_INV = 2.0
_NBUF = 4
_NSPLIT = 2
_NOUT = 4
_R = 4


def _u_copy(u_hbm, buf, sems, row, slot, j):
    n = u_hbm.shape[1] // _NSPLIT
    return pltpu.make_async_copy(
        u_hbm.at[row, pl.ds(j * n, n), :],
        buf.at[slot, pl.ds(j * n, n), :],
        sems.at[slot, j],
    )


def _body(l_hbm, u_hbm, o_hbm, buf, lbuf, obuf, sems, lsems, osems):
    B = u_hbm.shape[0]
    K = u_hbm.shape[1] // _R

    for r in range(_NBUF):
        for j in range(_NSPLIT):
            _u_copy(u_hbm, buf, sems, r, r, j).start(priority=j % 2)
        pltpu.make_async_copy(l_hbm.at[r], lbuf.at[r], lsems.at[r]).start()

    def step(b, carry):
        slot = jax.lax.rem(b, _NBUF)
        for j in range(_NSPLIT):
            _u_copy(u_hbm, buf, sems, b, slot, j).wait()
        pltpu.make_async_copy(l_hbm.at[b], lbuf.at[slot], lsems.at[slot]).wait()

        a = jnp.exp(lbuf[slot] * _TAU_INV)
        aa = jnp.concatenate([a] * K, axis=0)
        t = jnp.log(buf[slot])
        aw = aa / (t * t)
        sp = jnp.sum(aw, axis=1, keepdims=True)
        s = jnp.sum(sp.reshape(K, _R), axis=1, keepdims=True)
        r112 = jnp.repeat(1.0 / s, _R, axis=0)
        x = aw * r112
        x = jnp.maximum(x[0:56], x[56:112])
        x = jnp.maximum(x[0:28], x[28:56])
        g = x[0:4]
        for i in range(1, 7):
            g = jnp.maximum(g, x[4 * i:4 * i + 4])

        oslot = jax.lax.rem(b, _NOUT)

        @pl.when(b >= _NOUT)
        def _drain_prev():
            pltpu.make_async_copy(
                obuf.at[oslot], o_hbm.at[b - _NOUT], osems.at[oslot]).wait()

        obuf[oslot] = g
        pltpu.make_async_copy(
            obuf.at[oslot], o_hbm.at[b], osems.at[oslot]).start()

        b2 = b + _NBUF

        @pl.when(b2 < B)
        def _refill():
            slot2 = jax.lax.rem(b2, _NBUF)
            for j in range(_NSPLIT):
                _u_copy(u_hbm, buf, sems, b2, slot2, j).start(priority=j % 2)
            pltpu.make_async_copy(
                l_hbm.at[b2], lbuf.at[slot2], lsems.at[slot2]).start()

        return carry

    jax.lax.fori_loop(0, B, step, 0)

    for t in range(_NOUT):
        row = B - _NOUT + t
        pltpu.make_async_copy(
            obuf.at[row % _NOUT], o_hbm.at[row],
            osems.at[row % _NOUT]).wait()


def kernel(logits, uniform):
    B, K, D = uniform.shape
    Dc = D // _R
    out = pl.pallas_call(
        _body,
        in_specs=[
            pl.BlockSpec(memory_space=pltpu.HBM),
            pl.BlockSpec(memory_space=pltpu.HBM),
        ],
        out_specs=pl.BlockSpec(memory_space=pltpu.HBM),
        out_shape=jax.ShapeDtypeStruct((B, _R, Dc), jnp.float32),
        scratch_shapes=[
            pltpu.VMEM((_NBUF, K * _R, Dc), jnp.float32),
            pltpu.VMEM((_NBUF, _R, Dc), jnp.float32),
            pltpu.VMEM((_NOUT, _R, Dc), jnp.float32),
            pltpu.SemaphoreType.DMA((_NBUF, _NSPLIT)),
            pltpu.SemaphoreType.DMA((_NBUF,)),
            pltpu.SemaphoreType.DMA((_NOUT,)),
        ],
    )(logits.reshape(B, _R, Dc), uniform.reshape(B, K * _R, Dc))
    return out.reshape(B, D)
